# Initial kernel scaffold; baseline (speedup 1.0000x reference)
import jax
import jax.numpy as jnp
from jax.experimental import pallas as pl


def kernel(x, edge_index, W_theta, W_phi):
    raise NotImplementedError("write your pallas kernel here")


# speedup vs baseline: 3.1068x; 3.1068x over previous
import functools

import jax
import jax.numpy as jnp
from jax import lax
from jax.experimental import pallas as pl
from jax.experimental.pallas import tpu as pltpu
from jax.experimental.pallas import tpu_sc as plsc

_NUM_CORES = 2
_NUM_SUBCORES = 16
_NUM_TILES = _NUM_CORES * _NUM_SUBCORES
_LANES = 16


def _matmul_theta(x, W_theta):
    n, d_in = x.shape
    d_out = W_theta.shape[0]

    def body(w_ref, x_ref, o_ref):
        o_ref[...] = lax.dot_general(
            w_ref[...], x_ref[...],
            dimension_numbers=(((1,), (1,)), ((), ())),
            preferred_element_type=jnp.float32)

    return pl.pallas_call(
        body,
        out_shape=jax.ShapeDtypeStruct((d_out, n), jnp.float32),
    )(W_theta, x)


def _matmul_phi(yT, mT, W_phi):
    d_out, n = yT.shape

    def body(y_ref, m_ref, w_ref, o_ref):
        m = m_ref[...]
        z = jnp.where(jnp.isposinf(m), 0.0, y_ref[...] - m)
        o_ref[...] = lax.dot_general(
            z, w_ref[...],
            dimension_numbers=(((0,), (1,)), ((), ())),
            preferred_element_type=jnp.float32)

    return pl.pallas_call(
        body,
        out_shape=jax.ShapeDtypeStruct((n, d_out), jnp.float32),
    )(yT, mT, W_phi)


def _segment_min(yT_flat, rows, cols, d_out, n):
    e = rows.shape[0]
    cpt = d_out // _NUM_TILES
    ch = 6400
    assert e % ch == 0 and ch % _LANES == 0
    nv = ch // _LANES

    mesh = plsc.VectorSubcoreMesh(
        core_axis_name="c", subcore_axis_name="s")

    @functools.partial(
        pl.kernel,
        out_type=jax.ShapeDtypeStruct((d_out * n,), jnp.float32),
        mesh=mesh,
        compiler_params=pltpu.CompilerParams(needs_layout_passes=False),
        scratch_types=[
            pltpu.VMEM((cpt * n,), jnp.float32),
            pltpu.VMEM((cpt * n,), jnp.float32),
            pltpu.VMEM((ch,), jnp.int32),
            pltpu.VMEM((ch,), jnp.int32),
        ],
    )
    def k(yT_hbm, rows_hbm, cols_hbm, out_hbm, ycols, tbl, rbuf, cbuf):
        wid = lax.axis_index("s") * _NUM_CORES + lax.axis_index("c")
        base = wid * (cpt * n)
        pltpu.sync_copy(yT_hbm.at[pl.ds(base, cpt * n)], ycols)

        inf_v = jnp.full((_LANES,), jnp.inf, dtype=jnp.float32)

        def init_body(j, _):
            tbl[pl.ds(j * _LANES, _LANES)] = inf_v
            return 0
        lax.fori_loop(0, (cpt * n) // _LANES, init_body, 0)

        def chunk_body(ci, _):
            pltpu.sync_copy(rows_hbm.at[pl.ds(ci * ch, ch)], rbuf)
            pltpu.sync_copy(cols_hbm.at[pl.ds(ci * ch, ch)], cbuf)

            def vec_body(v, _):
                rws = rbuf[pl.ds(v * _LANES, _LANES)]
                cls = cbuf[pl.ds(v * _LANES, _LANES)]
                cnt, _last = plsc.scan_count(rws)
                first = cnt == 1
                yvs = [plsc.load_gather(ycols, [cls + d * n])
                       for d in range(cpt)]
                for d in range(cpt):
                    idx = rws + d * n
                    cur = plsc.load_gather(tbl, [idx])
                    plsc.store_scatter(
                        tbl, [idx], jnp.minimum(cur, yvs[d]), mask=first)
                pending = jnp.logical_not(first)

                @pl.when(jnp.any(pending))
                def _():
                    def wbody(p):
                        cnt_p, _ = plsc.scan_count(rws, mask=p)
                        wm = jnp.logical_and(p, cnt_p == 1)
                        for d in range(cpt):
                            idx = rws + d * n
                            cur = plsc.load_gather(tbl, [idx])
                            plsc.store_scatter(
                                tbl, [idx], jnp.minimum(cur, yvs[d]),
                                mask=wm)
                        return jnp.logical_and(p, jnp.logical_not(wm))
                    lax.while_loop(lambda p: jnp.any(p), wbody, pending)
                return 0
            lax.fori_loop(0, nv, vec_body, 0)
            return 0
        lax.fori_loop(0, e // ch, chunk_body, 0)

        pltpu.sync_copy(tbl, out_hbm.at[pl.ds(base, cpt * n)])

    return k(yT_flat, rows, cols)


def kernel(x, edge_index, W_theta, W_phi):
    n, _ = x.shape
    d_out = W_theta.shape[0]
    ei = edge_index.astype(jnp.int32)
    rows = ei[0]
    cols = ei[1]
    yT = _matmul_theta(x, W_theta)
    mT_flat = _segment_min(yT.reshape(-1), rows, cols, d_out, n)
    mT = mT_flat.reshape(d_out, n)
    return _matmul_phi(yT, mT, W_phi)


# speedup vs baseline: 3.3584x; 1.0810x over previous
import functools

import jax
import jax.numpy as jnp
from jax import lax
from jax.experimental import pallas as pl
from jax.experimental.pallas import tpu as pltpu
from jax.experimental.pallas import tpu_sc as plsc

_NUM_CORES = 2
_NUM_SUBCORES = 16
_NUM_TILES = _NUM_CORES * _NUM_SUBCORES
_LANES = 16


def _matmul_theta(x, W_theta):
    n, d_in = x.shape
    d_out = W_theta.shape[0]

    def body(w_ref, x_ref, o_ref):
        o_ref[...] = lax.dot_general(
            w_ref[...], x_ref[...],
            dimension_numbers=(((1,), (1,)), ((), ())),
            preferred_element_type=jnp.float32)

    return pl.pallas_call(
        body,
        out_shape=jax.ShapeDtypeStruct((d_out, n), jnp.float32),
    )(W_theta, x)


def _matmul_phi(yT, mT, W_phi):
    d_out, n = yT.shape

    def body(y_ref, m_ref, w_ref, o_ref):
        m = m_ref[...]
        z = jnp.where(jnp.isposinf(m), 0.0, y_ref[...] - m)
        o_ref[...] = lax.dot_general(
            z, w_ref[...],
            dimension_numbers=(((0,), (1,)), ((), ())),
            preferred_element_type=jnp.float32)

    return pl.pallas_call(
        body,
        out_shape=jax.ShapeDtypeStruct((n, d_out), jnp.float32),
    )(yT, mT, W_phi)


def _segment_min(yT_flat, rows, cols, d_out, n):
    e = rows.shape[0]
    cpt = d_out // _NUM_TILES
    ch = 6400
    unroll = 8
    assert e % ch == 0 and ch % (_LANES * unroll) == 0
    nv = ch // _LANES

    mesh = plsc.VectorSubcoreMesh(
        core_axis_name="c", subcore_axis_name="s")

    @functools.partial(
        pl.kernel,
        out_type=jax.ShapeDtypeStruct((d_out * n,), jnp.float32),
        mesh=mesh,
        compiler_params=pltpu.CompilerParams(needs_layout_passes=False),
        scratch_types=[
            pltpu.VMEM((cpt * n,), jnp.float32),
            pltpu.VMEM((cpt * n,), jnp.float32),
            pltpu.VMEM((ch,), jnp.int32),
            pltpu.VMEM((ch,), jnp.int32),
            pltpu.VMEM((ch,), jnp.int32),
            pltpu.VMEM((ch,), jnp.int32),
        ],
    )
    def k(yT_hbm, rows_hbm, cols_hbm, out_hbm,
          ycols, tbl, rbuf, cbuf, spr, spc):
        wid = lax.axis_index("s") * _NUM_CORES + lax.axis_index("c")
        base = wid * (cpt * n)
        pltpu.sync_copy(yT_hbm.at[pl.ds(base, cpt * n)], ycols)

        inf_v = jnp.full((_LANES,), jnp.inf, dtype=jnp.float32)

        def init_body(j, _):
            tbl[pl.ds(j * _LANES, _LANES)] = inf_v
            return 0
        lax.fori_loop(0, (cpt * n) // _LANES, init_body, 0)

        lane_iota = lax.iota(jnp.int32, _LANES)

        def chunk_body(ci, _):
            pltpu.sync_copy(rows_hbm.at[pl.ds(ci * ch, ch)], rbuf)
            pltpu.sync_copy(cols_hbm.at[pl.ds(ci * ch, ch)], cbuf)

            def grp_body(gi, off):
                for u in range(unroll):
                    v = gi * unroll + u
                    rws = rbuf[pl.ds(v * _LANES, _LANES)]
                    cls = cbuf[pl.ds(v * _LANES, _LANES)]
                    cnt, _last = plsc.scan_count(rws)
                    first = cnt == 1
                    for d in range(cpt):
                        idx = rws + d * n
                        yv = plsc.load_gather(ycols, [cls + d * n])
                        cur = plsc.load_gather(tbl, [idx])
                        plsc.store_scatter(
                            tbl, [idx], jnp.minimum(cur, yv), mask=first)
                    pending = jnp.logical_not(first)
                    pend_i = pending.astype(jnp.int32)
                    pos = off + lax.cumsum(pend_i) - 1
                    plsc.store_scatter(spr, [pos], rws, mask=pending)
                    plsc.store_scatter(spc, [pos], cls, mask=pending)
                    off = off + plsc.all_reduce_population_count(pending)
                return off
            off = lax.fori_loop(
                0, nv // unroll, grp_body,
                jnp.zeros((_LANES,), jnp.int32))
            ns = jnp.max(off)

            @pl.when(ns > 0)
            def _():
                def sgrp(g, _):
                    rws = spr[pl.ds(g * _LANES, _LANES)]
                    cls = spc[pl.ds(g * _LANES, _LANES)]
                    valid = (lane_iota + g * _LANES) < ns
                    yvs = [plsc.load_gather(ycols, [cls + d * n],
                                            mask=valid)
                           for d in range(cpt)]

                    def wbody(p):
                        cnt_p, _ = plsc.scan_count(rws, mask=p)
                        wm = jnp.logical_and(p, cnt_p == 1)
                        for d in range(cpt):
                            idx = rws + d * n
                            cur = plsc.load_gather(tbl, [idx], mask=wm)
                            plsc.store_scatter(
                                tbl, [idx], jnp.minimum(cur, yvs[d]),
                                mask=wm)
                        return jnp.logical_and(p, jnp.logical_not(wm))
                    lax.while_loop(lambda p: jnp.any(p), wbody, valid)
                    return 0
                lax.fori_loop(0, (ns + _LANES - 1) // _LANES, sgrp, 0)
            return 0
        lax.fori_loop(0, e // ch, chunk_body, 0)

        pltpu.sync_copy(tbl, out_hbm.at[pl.ds(base, cpt * n)])

    return k(yT_flat, rows, cols)


def kernel(x, edge_index, W_theta, W_phi):
    n, _ = x.shape
    d_out = W_theta.shape[0]
    ei = edge_index.astype(jnp.int32)
    rows = ei[0]
    cols = ei[1]
    yT = _matmul_theta(x, W_theta)
    mT_flat = _segment_min(yT.reshape(-1), rows, cols, d_out, n)
    mT = mT_flat.reshape(d_out, n)
    return _matmul_phi(yT, mT, W_phi)


# speedup vs baseline: 3.3935x; 1.0104x over previous
import functools

import jax
import jax.numpy as jnp
from jax import lax
from jax.experimental import pallas as pl
from jax.experimental.pallas import tpu as pltpu
from jax.experimental.pallas import tpu_sc as plsc

_NUM_CORES = 2
_NUM_SUBCORES = 16
_NUM_TILES = _NUM_CORES * _NUM_SUBCORES
_LANES = 16


def _matmul_theta(x, W_theta):
    n, d_in = x.shape
    d_out = W_theta.shape[0]

    def body(w_ref, x_ref, o_ref):
        o_ref[...] = lax.dot_general(
            w_ref[...], x_ref[...],
            dimension_numbers=(((1,), (1,)), ((), ())),
            preferred_element_type=jnp.float32)

    return pl.pallas_call(
        body,
        out_shape=jax.ShapeDtypeStruct((d_out, n), jnp.float32),
    )(W_theta, x)


def _matmul_phi(yT, mT, W_phi):
    d_out, n = yT.shape

    def body(y_ref, m_ref, w_ref, o_ref):
        m = m_ref[...]
        z = jnp.where(jnp.isposinf(m), 0.0, y_ref[...] - m)
        o_ref[...] = lax.dot_general(
            z, w_ref[...],
            dimension_numbers=(((0,), (1,)), ((), ())),
            preferred_element_type=jnp.float32)

    return pl.pallas_call(
        body,
        out_shape=jax.ShapeDtypeStruct((n, d_out), jnp.float32),
    )(yT, mT, W_phi)


def _segment_min(yT_flat, rows, cols, d_out, n):
    e = rows.shape[0]
    cpt = d_out // _NUM_TILES
    ch = 6400
    unroll = 8
    assert e % ch == 0 and ch % (_LANES * unroll) == 0
    nv = ch // _LANES

    mesh = plsc.VectorSubcoreMesh(
        core_axis_name="c", subcore_axis_name="s")

    @functools.partial(
        pl.kernel,
        out_type=jax.ShapeDtypeStruct((d_out * n,), jnp.float32),
        mesh=mesh,
        compiler_params=pltpu.CompilerParams(needs_layout_passes=False),
        scratch_types=[
            pltpu.VMEM((cpt * n,), jnp.float32),
            *[pltpu.VMEM((n,), jnp.float32) for _ in range(cpt)],
            pltpu.VMEM((ch,), jnp.int32),
            pltpu.VMEM((ch,), jnp.int32),
            pltpu.VMEM((ch,), jnp.int32),
            pltpu.VMEM((ch,), jnp.int32),
        ],
    )
    def k(yT_hbm, rows_hbm, cols_hbm, out_hbm,
          ycols, *rest):
        tbls = rest[:cpt]
        rbuf, cbuf, spr, spc = rest[cpt:]
        wid = lax.axis_index("s") * _NUM_CORES + lax.axis_index("c")
        base = wid * (cpt * n)
        pltpu.sync_copy(yT_hbm.at[pl.ds(base, cpt * n)], ycols)

        inf_v = jnp.full((_LANES,), jnp.inf, dtype=jnp.float32)

        def init_body(j, _):
            for d in range(cpt):
                tbls[d][pl.ds(j * _LANES, _LANES)] = inf_v
            return 0
        lax.fori_loop(0, n // _LANES, init_body, 0)

        lane_iota = lax.iota(jnp.int32, _LANES)

        def chunk_body(ci, _):
            pltpu.sync_copy(rows_hbm.at[pl.ds(ci * ch, ch)], rbuf)
            pltpu.sync_copy(cols_hbm.at[pl.ds(ci * ch, ch)], cbuf)

            def grp_body(gi, off):
                for u in range(unroll):
                    v = gi * unroll + u
                    rws = rbuf[pl.ds(v * _LANES, _LANES)]
                    cls = cbuf[pl.ds(v * _LANES, _LANES)]
                    cnt, _last = plsc.scan_count(rws)
                    first = cnt == 1
                    for d in range(cpt):
                        yv = plsc.load_gather(ycols, [cls + d * n])
                        cur = plsc.load_gather(tbls[d], [rws])
                        plsc.store_scatter(
                            tbls[d], [rws], jnp.minimum(cur, yv),
                            mask=first)
                    pending = jnp.logical_not(first)
                    pend_i = pending.astype(jnp.int32)
                    pos = off + lax.cumsum(pend_i) - 1
                    plsc.store_scatter(spr, [pos], rws, mask=pending)
                    plsc.store_scatter(spc, [pos], cls, mask=pending)
                    off = off + plsc.all_reduce_population_count(pending)
                return off
            off = lax.fori_loop(
                0, nv // unroll, grp_body,
                jnp.zeros((_LANES,), jnp.int32))
            ns = jnp.max(off)

            @pl.when(ns > 0)
            def _():
                def sgrp(g, _):
                    rws = spr[pl.ds(g * _LANES, _LANES)]
                    cls = spc[pl.ds(g * _LANES, _LANES)]
                    valid = (lane_iota + g * _LANES) < ns
                    yvs = [plsc.load_gather(ycols, [cls + d * n],
                                            mask=valid)
                           for d in range(cpt)]

                    def wbody(p):
                        cnt_p, _ = plsc.scan_count(rws, mask=p)
                        wm = jnp.logical_and(p, cnt_p == 1)
                        for d in range(cpt):
                            cur = plsc.load_gather(tbls[d], [rws],
                                                   mask=wm)
                            plsc.store_scatter(
                                tbls[d], [rws], jnp.minimum(cur, yvs[d]),
                                mask=wm)
                        return jnp.logical_and(p, jnp.logical_not(wm))
                    lax.while_loop(lambda p: jnp.any(p), wbody, valid)
                    return 0
                lax.fori_loop(0, (ns + _LANES - 1) // _LANES, sgrp, 0)
            return 0
        lax.fori_loop(0, e // ch, chunk_body, 0)

        for d in range(cpt):
            pltpu.sync_copy(tbls[d], out_hbm.at[pl.ds(base + d * n, n)])

    return k(yT_flat, rows, cols)


def kernel(x, edge_index, W_theta, W_phi):
    n, _ = x.shape
    d_out = W_theta.shape[0]
    ei = edge_index.astype(jnp.int32)
    rows = ei[0]
    cols = ei[1]
    yT = _matmul_theta(x, W_theta)
    mT_flat = _segment_min(yT.reshape(-1), rows, cols, d_out, n)
    mT = mT_flat.reshape(d_out, n)
    return _matmul_phi(yT, mT, W_phi)


# speedup vs baseline: 5.8326x; 1.7188x over previous
import functools

import jax
import jax.numpy as jnp
from jax import lax
from jax.experimental import pallas as pl
from jax.experimental.pallas import tpu as pltpu
from jax.experimental.pallas import tpu_sc as plsc

_NUM_CORES = 2
_NUM_SUBCORES = 16
_NUM_TILES = _NUM_CORES * _NUM_SUBCORES
_LANES = 16


def _matmul_theta(x, W_theta):
    n, d_in = x.shape
    d_out = W_theta.shape[0]

    def body(w_ref, x_ref, o_ref):
        o_ref[...] = lax.dot_general(
            w_ref[...], x_ref[...],
            dimension_numbers=(((1,), (1,)), ((), ())),
            preferred_element_type=jnp.float32)

    return pl.pallas_call(
        body,
        out_shape=jax.ShapeDtypeStruct((d_out, n), jnp.float32),
    )(W_theta, x)


def _matmul_phi(yT, mT, W_phi):
    d_out, n = yT.shape

    def body(y_ref, m_ref, w_ref, o_ref):
        m = m_ref[...]
        z = jnp.where(jnp.isposinf(m), 0.0, y_ref[...] - m)
        o_ref[...] = lax.dot_general(
            z, w_ref[...],
            dimension_numbers=(((0,), (1,)), ((), ())),
            preferred_element_type=jnp.float32)

    return pl.pallas_call(
        body,
        out_shape=jax.ShapeDtypeStruct((n, d_out), jnp.float32),
    )(yT, mT, W_phi)


def _segment_min(yT_flat, rows, cols, d_out, n):
    e = rows.shape[0]
    cpt = d_out // _NUM_TILES
    ch = 6400
    unroll = 8
    assert e % ch == 0 and ch % (_LANES * unroll) == 0
    nv = ch // _LANES

    mesh = plsc.VectorSubcoreMesh(
        core_axis_name="c", subcore_axis_name="s")

    @functools.partial(
        pl.kernel,
        out_type=jax.ShapeDtypeStruct((d_out * n,), jnp.float32),
        mesh=mesh,
        compiler_params=pltpu.CompilerParams(needs_layout_passes=False),
        scratch_types=[
            pltpu.VMEM((cpt * n,), jnp.float32),
            *[pltpu.VMEM((n,), jnp.float32) for _ in range(cpt)],
            pltpu.VMEM((ch,), jnp.int32),
            pltpu.VMEM((ch,), jnp.int32),
            pltpu.VMEM((ch,), jnp.int32),
            pltpu.VMEM((ch,), jnp.int32),
        ],
    )
    def k(yT_hbm, rows_hbm, cols_hbm, out_hbm,
          ycols, *rest):
        tbls = rest[:cpt]
        rbuf, cbuf, spr, spc = rest[cpt:]
        wid = lax.axis_index("s") * _NUM_CORES + lax.axis_index("c")
        base = wid * (cpt * n)
        pltpu.sync_copy(yT_hbm.at[pl.ds(base, cpt * n)], ycols)

        inf_v = jnp.full((_LANES,), jnp.inf, dtype=jnp.float32)

        def init_body(j, _):
            for d in range(cpt):
                tbls[d][pl.ds(j * _LANES, _LANES)] = inf_v
            return 0
        lax.fori_loop(0, n // _LANES, init_body, 0)

        lane_iota = lax.iota(jnp.int32, _LANES)

        def chunk_body(ci, _):
            pltpu.sync_copy(rows_hbm.at[pl.ds(ci * ch, ch)], rbuf)
            pltpu.sync_copy(cols_hbm.at[pl.ds(ci * ch, ch)], cbuf)

            def grp_body(gi, off):
                rws_l, first_l, yvs_l = [], [], []
                for u in range(unroll):
                    v = gi * unroll + u
                    rws = rbuf[pl.ds(v * _LANES, _LANES)]
                    cls = cbuf[pl.ds(v * _LANES, _LANES)]
                    cnt, _last = plsc.scan_count(rws)
                    rws_l.append(rws)
                    first_l.append(cnt == 1)
                    yvs_l.append([plsc.load_gather(ycols, [cls + d * n])
                                  for d in range(cpt)])
                for u in range(unroll):
                    curs = [plsc.load_gather(tbls[d], [rws_l[u]])
                            for d in range(cpt)]
                    for d in range(cpt):
                        plsc.store_scatter(
                            tbls[d], [rws_l[u]],
                            jnp.minimum(curs[d], yvs_l[u][d]),
                            mask=first_l[u])
                for u in range(unroll):
                    v = gi * unroll + u
                    cls = cbuf[pl.ds(v * _LANES, _LANES)]
                    pending = jnp.logical_not(first_l[u])
                    pend_i = pending.astype(jnp.int32)
                    pos = off + lax.cumsum(pend_i) - 1
                    plsc.store_scatter(spr, [pos], rws_l[u], mask=pending)
                    plsc.store_scatter(spc, [pos], cls, mask=pending)
                    off = off + plsc.all_reduce_population_count(pending)
                return off
            off = lax.fori_loop(
                0, nv // unroll, grp_body,
                jnp.zeros((_LANES,), jnp.int32))
            ns = jnp.max(off)

            @pl.when(ns > 0)
            def _():
                def sgrp(g, _):
                    rws = spr[pl.ds(g * _LANES, _LANES)]
                    cls = spc[pl.ds(g * _LANES, _LANES)]
                    valid = (lane_iota + g * _LANES) < ns
                    yvs = [plsc.load_gather(ycols, [cls + d * n],
                                            mask=valid)
                           for d in range(cpt)]

                    def wbody(p):
                        cnt_p, _ = plsc.scan_count(rws, mask=p)
                        wm = jnp.logical_and(p, cnt_p == 1)
                        for d in range(cpt):
                            cur = plsc.load_gather(tbls[d], [rws],
                                                   mask=wm)
                            plsc.store_scatter(
                                tbls[d], [rws], jnp.minimum(cur, yvs[d]),
                                mask=wm)
                        return jnp.logical_and(p, jnp.logical_not(wm))
                    lax.while_loop(lambda p: jnp.any(p), wbody, valid)
                    return 0
                lax.fori_loop(0, (ns + _LANES - 1) // _LANES, sgrp, 0)
            return 0
        lax.fori_loop(0, e // ch, chunk_body, 0)

        for d in range(cpt):
            pltpu.sync_copy(tbls[d], out_hbm.at[pl.ds(base + d * n, n)])

    return k(yT_flat, rows, cols)


def kernel(x, edge_index, W_theta, W_phi):
    n, _ = x.shape
    d_out = W_theta.shape[0]
    ei = edge_index.astype(jnp.int32)
    rows = ei[0]
    cols = ei[1]
    yT = _matmul_theta(x, W_theta)
    mT_flat = _segment_min(yT.reshape(-1), rows, cols, d_out, n)
    mT = mT_flat.reshape(d_out, n)
    return _matmul_phi(yT, mT, W_phi)


# speedup vs baseline: 7.2836x; 1.2488x over previous
import functools

import jax
import jax.numpy as jnp
from jax import lax
from jax.experimental import pallas as pl
from jax.experimental.pallas import tpu as pltpu
from jax.experimental.pallas import tpu_sc as plsc

_NUM_CORES = 2
_NUM_SUBCORES = 16
_NUM_TILES = _NUM_CORES * _NUM_SUBCORES
_LANES = 16
_INF_PACKED = 0x7F807F80


def _matmul_theta(x, W_theta_p):
    n, d_in = x.shape
    d_out = W_theta_p.shape[0]
    half = d_out // 2

    def body(w_ref, x_ref, y_ref, p_ref):
        y = lax.dot_general(
            w_ref[...], x_ref[...],
            dimension_numbers=(((1,), (1,)), ((), ())),
            preferred_element_type=jnp.float32)
        y_ref[...] = y
        hi = lax.bitcast_convert_type(
            y[:half].astype(jnp.bfloat16), jnp.uint16).astype(jnp.int32)
        lo = lax.bitcast_convert_type(
            y[half:].astype(jnp.bfloat16), jnp.uint16).astype(jnp.int32)
        p_ref[...] = (hi << 16) | lo

    return pl.pallas_call(
        body,
        out_shape=(
            jax.ShapeDtypeStruct((d_out, n), jnp.float32),
            jax.ShapeDtypeStruct((half, n), jnp.int32),
        ),
    )(W_theta_p, x)


def _matmul_phi(yT, m_packed, W_phi_p):
    d_out, n = yT.shape

    def body(y_ref, m_ref, w_ref, o_ref):
        mp = m_ref[...]
        m_hi = lax.bitcast_convert_type(
            mp & jnp.int32(-65536), jnp.float32)
        m_lo = lax.bitcast_convert_type(mp << 16, jnp.float32)
        m = jnp.concatenate([m_hi, m_lo], axis=0)
        z = jnp.where(jnp.isposinf(m), 0.0, y_ref[...] - m)
        o_ref[...] = lax.dot_general(
            z, w_ref[...],
            dimension_numbers=(((0,), (1,)), ((), ())),
            preferred_element_type=jnp.float32)

    return pl.pallas_call(
        body,
        out_shape=jax.ShapeDtypeStruct((n, d_out), jnp.float32),
    )(yT, m_packed, W_phi_p)


def _bf16_pair_min(a_i32, b_i32):
    a = plsc.bitcast(a_i32, jnp.bfloat16)
    b = plsc.bitcast(b_i32, jnp.bfloat16)
    return plsc.bitcast(jnp.minimum(a, b), jnp.int32)


def _segment_min(ypk_flat, rows, cols, npk, n):
    e = rows.shape[0]
    ppt = npk // _NUM_TILES
    ch = 6400
    unroll = 8
    assert e % ch == 0 and ch % (_LANES * unroll) == 0
    nv = ch // _LANES

    mesh = plsc.VectorSubcoreMesh(
        core_axis_name="c", subcore_axis_name="s")

    @functools.partial(
        pl.kernel,
        out_type=jax.ShapeDtypeStruct((npk * n,), jnp.int32),
        mesh=mesh,
        compiler_params=pltpu.CompilerParams(needs_layout_passes=False),
        scratch_types=[
            pltpu.VMEM((ppt * n,), jnp.int32),
            *[pltpu.VMEM((n,), jnp.int32) for _ in range(ppt)],
            pltpu.VMEM((ch,), jnp.int32),
            pltpu.VMEM((ch,), jnp.int32),
            pltpu.VMEM((ch,), jnp.int32),
            pltpu.VMEM((ch,), jnp.int32),
        ],
    )
    def k(ypk_hbm, rows_hbm, cols_hbm, out_hbm, ycols, *rest):
        tbls = rest[:ppt]
        rbuf, cbuf, spr, spc = rest[ppt:]
        wid = lax.axis_index("s") * _NUM_CORES + lax.axis_index("c")
        base = wid * (ppt * n)
        pltpu.sync_copy(ypk_hbm.at[pl.ds(base, ppt * n)], ycols)

        inf_v = jnp.full((_LANES,), _INF_PACKED, dtype=jnp.int32)

        def init_body(j, _):
            for p in range(ppt):
                tbls[p][pl.ds(j * _LANES, _LANES)] = inf_v
            return 0
        lax.fori_loop(0, n // _LANES, init_body, 0)

        lane_iota = lax.iota(jnp.int32, _LANES)

        def chunk_body(ci, _):
            pltpu.sync_copy(rows_hbm.at[pl.ds(ci * ch, ch)], rbuf)
            pltpu.sync_copy(cols_hbm.at[pl.ds(ci * ch, ch)], cbuf)

            def grp_body(gi, off):
                rws_l, first_l, yvs_l = [], [], []
                for u in range(unroll):
                    v = gi * unroll + u
                    rws = rbuf[pl.ds(v * _LANES, _LANES)]
                    cls = cbuf[pl.ds(v * _LANES, _LANES)]
                    cnt, _last = plsc.scan_count(rws)
                    rws_l.append(rws)
                    first_l.append(cnt == 1)
                    yvs_l.append([plsc.load_gather(ycols, [cls + p * n])
                                  for p in range(ppt)])
                for u in range(unroll):
                    curs = [plsc.load_gather(tbls[p], [rws_l[u]])
                            for p in range(ppt)]
                    for p in range(ppt):
                        plsc.store_scatter(
                            tbls[p], [rws_l[u]],
                            _bf16_pair_min(curs[p], yvs_l[u][p]),
                            mask=first_l[u])
                for u in range(unroll):
                    v = gi * unroll + u
                    cls = cbuf[pl.ds(v * _LANES, _LANES)]
                    pending = jnp.logical_not(first_l[u])
                    pend_i = pending.astype(jnp.int32)
                    pos = off + lax.cumsum(pend_i) - 1
                    plsc.store_scatter(spr, [pos], rws_l[u], mask=pending)
                    plsc.store_scatter(spc, [pos], cls, mask=pending)
                    off = off + plsc.all_reduce_population_count(pending)
                return off
            off = lax.fori_loop(
                0, nv // unroll, grp_body,
                jnp.zeros((_LANES,), jnp.int32))
            ns = jnp.max(off)

            @pl.when(ns > 0)
            def _():
                def sgrp(g, _):
                    rws = spr[pl.ds(g * _LANES, _LANES)]
                    cls = spc[pl.ds(g * _LANES, _LANES)]
                    valid = (lane_iota + g * _LANES) < ns
                    yvs = [plsc.load_gather(ycols, [cls + p * n],
                                            mask=valid)
                           for p in range(ppt)]

                    def wbody(pn):
                        cnt_p, _ = plsc.scan_count(rws, mask=pn)
                        wm = jnp.logical_and(pn, cnt_p == 1)
                        for p in range(ppt):
                            cur = plsc.load_gather(tbls[p], [rws],
                                                   mask=wm)
                            plsc.store_scatter(
                                tbls[p], [rws],
                                _bf16_pair_min(cur, yvs[p]),
                                mask=wm)
                        return jnp.logical_and(pn, jnp.logical_not(wm))
                    lax.while_loop(lambda pn: jnp.any(pn), wbody, valid)
                    return 0
                lax.fori_loop(0, (ns + _LANES - 1) // _LANES, sgrp, 0)
            return 0
        lax.fori_loop(0, e // ch, chunk_body, 0)

        for p in range(ppt):
            pltpu.sync_copy(tbls[p], out_hbm.at[pl.ds(base + p * n, n)])

    return k(ypk_flat, rows, cols)


def kernel(x, edge_index, W_theta, W_phi):
    n, _ = x.shape
    d_out = W_theta.shape[0]
    npk = d_out // 2
    W_theta_p = jnp.concatenate([W_theta[0::2], W_theta[1::2]], axis=0)
    W_phi_p = jnp.concatenate([W_phi[:, 0::2], W_phi[:, 1::2]], axis=1)
    ei = edge_index.astype(jnp.int32)
    rows = ei[0]
    cols = ei[1]
    yT, ypk = _matmul_theta(x, W_theta_p)
    m_packed = _segment_min(ypk.reshape(-1), rows, cols, npk, n)
    return _matmul_phi(yT, m_packed.reshape(npk, n), W_phi_p)


# speedup vs baseline: 11.0606x; 1.5186x over previous
import functools

import jax
import jax.numpy as jnp
from jax import lax
from jax.experimental import pallas as pl
from jax.experimental.pallas import tpu as pltpu
from jax.experimental.pallas import tpu_sc as plsc

_NUM_CORES = 2
_NUM_SUBCORES = 16
_NUM_TILES = _NUM_CORES * _NUM_SUBCORES
_LANES = 16
_INF_PACKED = 0x7F807F80


def _matmul_theta(x, W_theta_p):
    n, d_in = x.shape
    d_out = W_theta_p.shape[0]
    half = d_out // 2

    def body(w_ref, x_ref, y_ref, p_ref):
        y = lax.dot_general(
            w_ref[...], x_ref[...],
            dimension_numbers=(((1,), (1,)), ((), ())),
            preferred_element_type=jnp.float32)
        y_ref[...] = y
        hi = lax.bitcast_convert_type(
            y[:half].astype(jnp.bfloat16), jnp.uint16).astype(jnp.int32)
        lo = lax.bitcast_convert_type(
            y[half:].astype(jnp.bfloat16), jnp.uint16).astype(jnp.int32)
        p_ref[...] = (hi << 16) | lo

    return pl.pallas_call(
        body,
        out_shape=(
            jax.ShapeDtypeStruct((d_out, n), jnp.float32),
            jax.ShapeDtypeStruct((half, n), jnp.int32),
        ),
    )(W_theta_p, x)


def _matmul_phi(yT, m_packed, W_phi_p):
    d_out, n = yT.shape

    def body(y_ref, m_ref, w_ref, o_ref):
        mp = m_ref[...]
        m_hi = lax.bitcast_convert_type(
            mp & jnp.int32(-65536), jnp.float32)
        m_lo = lax.bitcast_convert_type(mp << 16, jnp.float32)
        m = jnp.concatenate([m_hi, m_lo], axis=0)
        z = jnp.where(jnp.isposinf(m), 0.0, y_ref[...] - m)
        o_ref[...] = lax.dot_general(
            z, w_ref[...],
            dimension_numbers=(((0,), (1,)), ((), ())),
            preferred_element_type=jnp.float32)

    return pl.pallas_call(
        body,
        out_shape=jax.ShapeDtypeStruct((n, d_out), jnp.float32),
    )(yT, m_packed, W_phi_p)


def _bf16_pair_min(a_i32, b_i32):
    a = plsc.bitcast(a_i32, jnp.bfloat16)
    b = plsc.bitcast(b_i32, jnp.bfloat16)
    return plsc.bitcast(jnp.minimum(a, b), jnp.int32)


def _segment_min(ypk_flat, rows, cols, npk, n):
    e = rows.shape[0]
    ppt = npk // _NUM_TILES
    ch = 6400
    unroll = 8
    assert e % ch == 0 and ch % (_LANES * unroll) == 0
    nv = ch // _LANES

    mesh = plsc.VectorSubcoreMesh(
        core_axis_name="c", subcore_axis_name="s")

    @functools.partial(
        pl.kernel,
        out_type=jax.ShapeDtypeStruct((npk * n,), jnp.int32),
        mesh=mesh,
        compiler_params=pltpu.CompilerParams(needs_layout_passes=False),
        scratch_types=[
            pltpu.VMEM((ppt * n,), jnp.int32),
            *[pltpu.VMEM((n,), jnp.int32) for _ in range(ppt)],
            pltpu.VMEM((ch,), jnp.int32),
            pltpu.VMEM((ch,), jnp.int32),
            pltpu.VMEM((ch,), jnp.int32),
            pltpu.VMEM((ch,), jnp.int32),
            pltpu.VMEM((ch,), jnp.int32),
            pltpu.VMEM((ch,), jnp.int32),
            pltpu.SemaphoreType.DMA,
            pltpu.SemaphoreType.DMA,
            pltpu.SemaphoreType.DMA,
            pltpu.SemaphoreType.DMA,
        ],
    )
    def k(ypk_hbm, rows_hbm, cols_hbm, out_hbm, ycols, *rest):
        tbls = rest[:ppt]
        (rbuf0, cbuf0, rbuf1, cbuf1, spr, spc,
         sr0, sc0, sr1, sc1) = rest[ppt:]
        wid = lax.axis_index("s") * _NUM_CORES + lax.axis_index("c")
        base = wid * (ppt * n)
        pltpu.sync_copy(ypk_hbm.at[pl.ds(base, ppt * n)], ycols)

        inf_v = jnp.full((_LANES,), _INF_PACKED, dtype=jnp.int32)

        def init_body(j, _):
            for p in range(ppt):
                tbls[p][pl.ds(j * _LANES, _LANES)] = inf_v
            return 0
        lax.fori_loop(0, n // _LANES, init_body, 0)

        lane_iota = lax.iota(jnp.int32, _LANES)
        nchunks = e // ch

        def process_chunk(rbuf, cbuf):
            def grp_body(gi, off):
                rws_l, cls_l, first_l, yvs_l = [], [], [], []
                for u in range(unroll):
                    v = gi * unroll + u
                    rws = rbuf[pl.ds(v * _LANES, _LANES)]
                    cls = cbuf[pl.ds(v * _LANES, _LANES)]
                    cnt, _last = plsc.scan_count(rws)
                    rws_l.append(rws)
                    cls_l.append(cls)
                    first_l.append(cnt == 1)
                    yvs_l.append([plsc.load_gather(ycols, [cls + p * n])
                                  for p in range(ppt)])
                for u in range(unroll):
                    curs = [plsc.load_gather(tbls[p], [rws_l[u]])
                            for p in range(ppt)]
                    for p in range(ppt):
                        plsc.store_scatter(
                            tbls[p], [rws_l[u]],
                            _bf16_pair_min(curs[p], yvs_l[u][p]),
                            mask=first_l[u])
                for u in range(unroll):
                    pending = jnp.logical_not(first_l[u])
                    pend_i = pending.astype(jnp.int32)
                    pos = off + lax.cumsum(pend_i) - 1
                    plsc.store_scatter(spr, [pos], rws_l[u], mask=pending)
                    plsc.store_scatter(spc, [pos], cls_l[u], mask=pending)
                    off = off + plsc.all_reduce_population_count(pending)
                return off
            off = lax.fori_loop(
                0, nv // unroll, grp_body,
                jnp.zeros((_LANES,), jnp.int32))
            ns = jnp.max(off)

            @pl.when(ns > 0)
            def _():
                def sgrp(g, _):
                    rws = spr[pl.ds(g * _LANES, _LANES)]
                    cls = spc[pl.ds(g * _LANES, _LANES)]
                    valid = (lane_iota + g * _LANES) < ns
                    yvs = [plsc.load_gather(ycols, [cls + p * n],
                                            mask=valid)
                           for p in range(ppt)]

                    def wbody(pn):
                        cnt_p, _ = plsc.scan_count(rws, mask=pn)
                        wm = jnp.logical_and(pn, cnt_p == 1)
                        for p in range(ppt):
                            cur = plsc.load_gather(tbls[p], [rws],
                                                   mask=wm)
                            plsc.store_scatter(
                                tbls[p], [rws],
                                _bf16_pair_min(cur, yvs[p]),
                                mask=wm)
                        return jnp.logical_and(pn, jnp.logical_not(wm))
                    lax.while_loop(lambda pn: jnp.any(pn), wbody, valid)
                    return 0
                lax.fori_loop(0, (ns + _LANES - 1) // _LANES, sgrp, 0)

        def start_copy(ci, rbuf, cbuf, sr, sc):
            pltpu.async_copy(rows_hbm.at[pl.ds(ci * ch, ch)], rbuf, sr)
            pltpu.async_copy(cols_hbm.at[pl.ds(ci * ch, ch)], cbuf, sc)

        def wait_copy(ci, rbuf, cbuf, sr, sc):
            pltpu.make_async_copy(
                rows_hbm.at[pl.ds(ci * ch, ch)], rbuf, sr).wait()
            pltpu.make_async_copy(
                cols_hbm.at[pl.ds(ci * ch, ch)], cbuf, sc).wait()

        start_copy(0, rbuf0, cbuf0, sr0, sc0)

        def pair_body(pi, _):
            ci0 = 2 * pi
            wait_copy(ci0, rbuf0, cbuf0, sr0, sc0)
            start_copy(ci0 + 1, rbuf1, cbuf1, sr1, sc1)
            process_chunk(rbuf0, cbuf0)
            wait_copy(ci0 + 1, rbuf1, cbuf1, sr1, sc1)
            start_copy((ci0 + 2) % nchunks, rbuf0, cbuf0, sr0, sc0)
            process_chunk(rbuf1, cbuf1)
            return 0
        assert nchunks % 2 == 0
        lax.fori_loop(0, nchunks // 2, pair_body, 0)
        wait_copy(0, rbuf0, cbuf0, sr0, sc0)

        for p in range(ppt):
            pltpu.sync_copy(tbls[p], out_hbm.at[pl.ds(base + p * n, n)])

    return k(ypk_flat, rows, cols)


def kernel(x, edge_index, W_theta, W_phi):
    n, _ = x.shape
    d_out = W_theta.shape[0]
    npk = d_out // 2
    W_theta_p = jnp.concatenate([W_theta[0::2], W_theta[1::2]], axis=0)
    W_phi_p = jnp.concatenate([W_phi[:, 0::2], W_phi[:, 1::2]], axis=1)
    ei = edge_index.astype(jnp.int32)
    rows = ei[0]
    cols = ei[1]
    yT, ypk = _matmul_theta(x, W_theta_p)
    m_packed = _segment_min(ypk.reshape(-1), rows, cols, npk, n)
    return _matmul_phi(yT, m_packed.reshape(npk, n), W_phi_p)
